# baseline (speedup 1.0000x reference)
import functools

import jax
import jax.numpy as jnp
from jax import lax
from jax.experimental import pallas as pl
from jax.experimental.pallas import tpu as pltpu
from jax.experimental.pallas import tpu_sc as plsc

B = 65536
NW = 32
BPW = B // NW
CB = 128
NCHUNK = BPW // CB
F = 42
K = 276
NKC = 18

_mesh = plsc.VectorSubcoreMesh(core_axis_name="c", subcore_axis_name="s")


@functools.partial(
    pl.kernel,
    mesh=_mesh,
    out_type=jax.ShapeDtypeStruct((B * K,), jnp.float32),
    compiler_params=pltpu.CompilerParams(needs_layout_passes=False),
    scratch_types=[
        pltpu.VMEM((NKC, 16), jnp.int32),
        pltpu.VMEM((CB * F,), jnp.float32),
        pltpu.VMEM((CB * K,), jnp.float32),
    ],
)
def _gather_kernel(x_hbm, gf_hbm, out_hbm, gf_v, in_v, out_v):
    wid = lax.axis_index("s") * 2 + lax.axis_index("c")
    base0 = wid * BPW
    pltpu.sync_copy(gf_hbm, gf_v)

    def chunk_body(ci, carry):
        base = base0 + ci * CB
        pltpu.sync_copy(x_hbm.at[pl.ds(base * F, CB * F)], in_v)

        def kc_body(c, carry2):
            gfvec = gf_v[c]
            start = jnp.minimum(c * 16, K - 16)

            def b_body(b, carry3):
                idx = gfvec + b * F
                vals = plsc.load_gather(in_v, [idx])
                out_v[pl.ds(b * K + start, 16)] = vals
                return carry3

            return lax.fori_loop(0, CB, b_body, carry2)

        lax.fori_loop(0, NKC, kc_body, 0)
        pltpu.sync_copy(out_v, out_hbm.at[pl.ds(base * K, CB * K)])
        return carry

    lax.fori_loop(0, NCHUNK, chunk_body, 0)


def _build_gf_chunks(groupings):
    gfl = groupings.reshape(-1).astype(jnp.int32)
    starts = jnp.minimum(16 * jnp.arange(NKC, dtype=jnp.int32), K - 16)
    return gfl[starts[:, None] + jnp.arange(16, dtype=jnp.int32)[None, :]]


def kernel(inputs, groupings):
    x = inputs.reshape(B * F)
    gfc = _build_gf_chunks(groupings)
    out = _gather_kernel(x, gfc)
    return out.reshape(B, 69, 4)


# speedup vs baseline: 29.2492x; 29.2492x over previous
import functools

import jax
import jax.numpy as jnp
from jax import lax
from jax.experimental import pallas as pl
from jax.experimental.pallas import tpu as pltpu
from jax.experimental.pallas import tpu_sc as plsc

B = 65536
NBLK = B // 128
NW = 32
K = 276
ROWS_OUT = 69 * NBLK * 4
RPT = ROWS_OUT // NW
CHUNK = 96
NCH = RPT // CHUNK
RC_PAD = 288

_mesh = plsc.VectorSubcoreMesh(core_axis_name="c", subcore_axis_name="s")


@functools.partial(
    pl.kernel,
    mesh=_mesh,
    out_type=jax.ShapeDtypeStruct((ROWS_OUT, 128), jnp.float32),
    compiler_params=pltpu.CompilerParams(needs_layout_passes=False),
    scratch_types=[
        pltpu.VMEM((RC_PAD,), jnp.int32),
        pltpu.VMEM((CHUNK,), jnp.int32),
        pltpu.VMEM((CHUNK, 128), jnp.float32),
        pltpu.SemaphoreType.DMA,
    ],
)
def _row_gather_kernel(xrows_hbm, rc_hbm, out_hbm, rc_v, idx_v, buf_v, sem):
    wid = lax.axis_index("s") * 2 + lax.axis_index("c")
    row0 = wid * RPT
    pltpu.sync_copy(rc_hbm, rc_v)
    lanes = jax.lax.iota(jnp.int32, 16)

    def chunk_body(ch, carry):
        obase = row0 + ch * CHUNK

        def build_body(p, carry2):
            o = obase + p * 16 + lanes
            i = lax.shift_right_logical(o, 11)
            k = lax.shift_left(i, 2) + (o & 3)
            bblk = lax.shift_right_logical(o, 2) & 511
            rc = plsc.load_gather(rc_v, [k])
            idx_v[pl.ds(p * 16, 16)] = rc + lax.shift_left(bblk, 3)
            return carry2

        lax.fori_loop(0, CHUNK // 16, build_body, 0)
        pltpu.async_copy(xrows_hbm.at[idx_v], buf_v, sem).wait()
        pltpu.sync_copy(buf_v, out_hbm.at[pl.ds(obase, CHUNK)])
        return carry

    lax.fori_loop(0, NCH, chunk_body, 0)


def _rc_table(groupings):
    gfl = groupings.reshape(-1).astype(jnp.int32)
    rc = (gfl // 7) * 4096 + gfl % 7
    return jnp.concatenate([rc, jnp.zeros((RC_PAD - K,), jnp.int32)])


def kernel(inputs, groupings):
    t = inputs.transpose(1, 0, 2).reshape(6, NBLK, 128, 7).transpose(0, 1, 3, 2)
    xrows = jnp.pad(t, ((0, 0), (0, 0), (0, 1), (0, 0))).reshape(6 * NBLK * 8, 128)
    out_rows = _row_gather_kernel(xrows, _rc_table(groupings))
    return out_rows.reshape(69, NBLK, 4, 128).transpose(1, 3, 0, 2).reshape(B, 69, 4)


# speedup vs baseline: 40.9707x; 1.4007x over previous
import functools

import jax
import jax.numpy as jnp
from jax import lax
from jax.experimental import pallas as pl
from jax.experimental.pallas import tpu as pltpu
from jax.experimental.pallas import tpu_sc as plsc

B = 65536
NBLK = B // 128
NW = 32
K = 276
ROWS_OUT = 69 * NBLK * 4
RPT = ROWS_OUT // NW
CHUNK = 64
NCH = RPT // CHUNK
NBUF = 3
RC_PAD = 288

_mesh = plsc.VectorSubcoreMesh(core_axis_name="c", subcore_axis_name="s")


@functools.partial(
    pl.kernel,
    mesh=_mesh,
    out_type=jax.ShapeDtypeStruct((ROWS_OUT, 128), jnp.float32),
    compiler_params=pltpu.CompilerParams(needs_layout_passes=False),
    scratch_types=[
        pltpu.VMEM((RC_PAD,), jnp.int32),
        *[pltpu.VMEM((CHUNK,), jnp.int32) for _ in range(NBUF)],
        *[pltpu.VMEM((CHUNK, 128), jnp.float32) for _ in range(NBUF)],
        *[pltpu.SemaphoreType.DMA for _ in range(2 * NBUF)],
    ],
)
def _row_gather_kernel(xrows_hbm, rc_hbm, out_hbm, rc_v, i0, i1, i2, b0, b1, b2,
                       g0, g1, g2, o0, o1, o2):
    idx_v, buf_v, gsem, osem = (i0, i1, i2), (b0, b1, b2), (g0, g1, g2), (o0, o1, o2)
    wid = lax.axis_index("s") * 2 + lax.axis_index("c")
    row0 = wid * RPT
    pltpu.sync_copy(rc_hbm, rc_v)
    lanes = jax.lax.iota(jnp.int32, 16)

    def build_idx(s, ch):
        obase = row0 + ch * CHUNK

        def build_body(p, carry2):
            o = obase + p * 16 + lanes
            i = lax.shift_right_logical(o, 11)
            k = lax.shift_left(i, 2) + (o & 3)
            bblk = lax.shift_right_logical(o, 2) & 511
            rc = plsc.load_gather(rc_v, [k])
            idx_v[s][pl.ds(p * 16, 16)] = rc + lax.shift_left(bblk, 3)
            return carry2

        lax.fori_loop(0, CHUNK // 16, build_body, 0)

    for s in range(NBUF):
        build_idx(s, jnp.int32(s))
        pltpu.async_copy(xrows_hbm.at[idx_v[s]], buf_v[s], gsem[s])

    def ring_body(p, carry):
        for s in range(NBUF):
            ch = p * NBUF + s
            obase = row0 + ch * CHUNK
            out_slice = out_hbm.at[pl.ds(obase, CHUNK)]
            pltpu.make_async_copy(xrows_hbm.at[idx_v[s]], buf_v[s], gsem[s]).wait()
            pltpu.async_copy(buf_v[s], out_slice, osem[s])

            @pl.when(ch + NBUF < NCH)
            def _():
                build_idx(s, ch + NBUF)
                pltpu.make_async_copy(buf_v[s], out_slice, osem[s]).wait()
                pltpu.async_copy(xrows_hbm.at[idx_v[s]], buf_v[s], gsem[s])

        return carry

    lax.fori_loop(0, NCH // NBUF, ring_body, 0)
    for s in range(NBUF):
        pltpu.make_async_copy(buf_v[s], out_hbm.at[pl.ds(row0, CHUNK)], osem[s]).wait()


def _rc_table(groupings):
    gfl = groupings.reshape(-1).astype(jnp.int32)
    rc = (gfl // 7) * 4096 + gfl % 7
    return jnp.concatenate([rc, jnp.zeros((RC_PAD - K,), jnp.int32)])


def kernel(inputs, groupings):
    t = inputs.transpose(1, 0, 2).reshape(6, NBLK, 128, 7).transpose(0, 1, 3, 2)
    xrows = jnp.pad(t, ((0, 0), (0, 0), (0, 1), (0, 0))).reshape(6 * NBLK * 8, 128)
    out_rows = _row_gather_kernel(xrows, _rc_table(groupings))
    return out_rows.reshape(69, NBLK, 4, 128).transpose(1, 3, 0, 2).reshape(B, 69, 4)


# speedup vs baseline: 41.0283x; 1.0014x over previous
import functools

import jax
import jax.numpy as jnp
from jax import lax
from jax.experimental import pallas as pl
from jax.experimental.pallas import tpu as pltpu
from jax.experimental.pallas import tpu_sc as plsc

B = 65536
NBLK = B // 128
NW = 32
K = 276
ROWS_OUT = 69 * NBLK * 4
RPT = ROWS_OUT // NW
CHUNK = 64
NCH = RPT // CHUNK
NBUF = 3
RC_PAD = 288

_mesh = plsc.VectorSubcoreMesh(core_axis_name="c", subcore_axis_name="s")


@functools.partial(
    pl.kernel,
    mesh=_mesh,
    out_type=jax.ShapeDtypeStruct((ROWS_OUT, 128), jnp.float32),
    compiler_params=pltpu.CompilerParams(needs_layout_passes=False),
    scratch_types=[
        pltpu.VMEM((RC_PAD,), jnp.int32),
        *[pltpu.VMEM((CHUNK,), jnp.int32) for _ in range(NBUF)],
        *[pltpu.VMEM((CHUNK, 128), jnp.float32) for _ in range(NBUF)],
        *[pltpu.SemaphoreType.DMA for _ in range(2 * NBUF)],
    ],
)
def _row_gather_kernel(xrows_hbm, rc_hbm, out_hbm, rc_v, i0, i1, i2, b0, b1, b2,
                       g0, g1, g2, o0, o1, o2):
    idx_v, buf_v, gsem, osem = (i0, i1, i2), (b0, b1, b2), (g0, g1, g2), (o0, o1, o2)
    wid = lax.axis_index("s") * 2 + lax.axis_index("c")
    row0 = wid * RPT
    pltpu.sync_copy(rc_hbm, rc_v)
    lanes = jax.lax.iota(jnp.int32, 16)

    def rc_body(p, carry):
        g = rc_v[pl.ds(p * 16, 16)]
        r = lax.div(g, 7)
        rc_v[pl.ds(p * 16, 16)] = lax.shift_left(r, 12) + (g - r * 7)
        return carry

    lax.fori_loop(0, RC_PAD // 16, rc_body, 0)

    def build_idx(s, ch):
        obase = row0 + ch * CHUNK

        def build_body(p, carry2):
            o = obase + p * 16 + lanes
            i = lax.shift_right_logical(o, 11)
            k = lax.shift_left(i, 2) + (o & 3)
            bblk = lax.shift_right_logical(o, 2) & 511
            rc = plsc.load_gather(rc_v, [k])
            idx_v[s][pl.ds(p * 16, 16)] = rc + lax.shift_left(bblk, 3)
            return carry2

        lax.fori_loop(0, CHUNK // 16, build_body, 0)

    for s in range(NBUF):
        build_idx(s, jnp.int32(s))
        pltpu.async_copy(xrows_hbm.at[idx_v[s]], buf_v[s], gsem[s])

    def ring_body(p, carry):
        for s in range(NBUF):
            ch = p * NBUF + s
            obase = row0 + ch * CHUNK
            out_slice = out_hbm.at[pl.ds(obase, CHUNK)]
            pltpu.make_async_copy(xrows_hbm.at[idx_v[s]], buf_v[s], gsem[s]).wait()
            pltpu.async_copy(buf_v[s], out_slice, osem[s])

            @pl.when(ch + NBUF < NCH)
            def _():
                build_idx(s, ch + NBUF)
                pltpu.make_async_copy(buf_v[s], out_slice, osem[s]).wait()
                pltpu.async_copy(xrows_hbm.at[idx_v[s]], buf_v[s], gsem[s])

        return carry

    lax.fori_loop(0, NCH // NBUF, ring_body, 0)
    for s in range(NBUF):
        pltpu.make_async_copy(buf_v[s], out_hbm.at[pl.ds(row0, CHUNK)], osem[s]).wait()


def _g_flat(groupings):
    gfl = groupings.reshape(-1).astype(jnp.int32)
    return jnp.pad(gfl, (0, RC_PAD - K))


def kernel(inputs, groupings):
    t = inputs.transpose(1, 0, 2).reshape(6, NBLK, 128, 7).transpose(0, 1, 3, 2)
    xrows = jnp.pad(t, ((0, 0), (0, 0), (0, 1), (0, 0))).reshape(6 * NBLK * 8, 128)
    out_rows = _row_gather_kernel(xrows, _g_flat(groupings))
    return out_rows.reshape(69, NBLK, 4, 128).transpose(1, 3, 0, 2).reshape(B, 69, 4)


# speedup vs baseline: 41.1448x; 1.0028x over previous
import functools

import jax
import jax.numpy as jnp
from jax import lax
from jax.experimental import pallas as pl
from jax.experimental.pallas import tpu as pltpu
from jax.experimental.pallas import tpu_sc as plsc

B = 65536
NBLK = B // 128
NW = 32
K = 276
ROWS_OUT = 69 * NBLK * 4
RPT = ROWS_OUT // NW
CHUNK = 48
NCH = RPT // CHUNK
NBUF = 4
RC_PAD = 288

_mesh = plsc.VectorSubcoreMesh(core_axis_name="c", subcore_axis_name="s")


@functools.partial(
    pl.kernel,
    mesh=_mesh,
    out_type=jax.ShapeDtypeStruct((ROWS_OUT, 128), jnp.float32),
    compiler_params=pltpu.CompilerParams(needs_layout_passes=False),
    scratch_types=[
        pltpu.VMEM((RC_PAD,), jnp.int32),
        *[pltpu.VMEM((CHUNK,), jnp.int32) for _ in range(NBUF)],
        *[pltpu.VMEM((CHUNK, 128), jnp.float32) for _ in range(NBUF)],
        *[pltpu.SemaphoreType.DMA for _ in range(2 * NBUF)],
    ],
)
def _row_gather_kernel(xrows_hbm, rc_hbm, out_hbm, rc_v, *ring):
    idx_v = ring[0:NBUF]
    buf_v = ring[NBUF:2 * NBUF]
    gsem = ring[2 * NBUF:3 * NBUF]
    osem = ring[3 * NBUF:4 * NBUF]
    wid = lax.axis_index("s") * 2 + lax.axis_index("c")
    row0 = wid * RPT
    pltpu.sync_copy(rc_hbm, rc_v)
    lanes = jax.lax.iota(jnp.int32, 16)

    def rc_body(p, carry):
        g = rc_v[pl.ds(p * 16, 16)]
        r = lax.div(g, 7)
        rc_v[pl.ds(p * 16, 16)] = lax.shift_left(r, 12) + (g - r * 7)
        return carry

    lax.fori_loop(0, RC_PAD // 16, rc_body, 0)

    def build_idx(s, ch):
        obase = row0 + ch * CHUNK

        def build_body(p, carry2):
            o = obase + p * 16 + lanes
            i = lax.shift_right_logical(o, 11)
            k = lax.shift_left(i, 2) + (o & 3)
            bblk = lax.shift_right_logical(o, 2) & 511
            rc = plsc.load_gather(rc_v, [k])
            idx_v[s][pl.ds(p * 16, 16)] = rc + lax.shift_left(bblk, 3)
            return carry2

        lax.fori_loop(0, CHUNK // 16, build_body, 0)

    for s in range(NBUF):
        build_idx(s, jnp.int32(s))
        pltpu.async_copy(xrows_hbm.at[idx_v[s]], buf_v[s], gsem[s])

    def ring_body(p, carry):
        for s in range(NBUF):
            ch = p * NBUF + s
            obase = row0 + ch * CHUNK
            out_slice = out_hbm.at[pl.ds(obase, CHUNK)]
            pltpu.make_async_copy(xrows_hbm.at[idx_v[s]], buf_v[s], gsem[s]).wait()
            pltpu.async_copy(buf_v[s], out_slice, osem[s])

            @pl.when(ch + NBUF < NCH)
            def _():
                build_idx(s, ch + NBUF)
                pltpu.make_async_copy(buf_v[s], out_slice, osem[s]).wait()
                pltpu.async_copy(xrows_hbm.at[idx_v[s]], buf_v[s], gsem[s])

        return carry

    lax.fori_loop(0, NCH // NBUF, ring_body, 0)
    for s in range(NBUF):
        pltpu.make_async_copy(buf_v[s], out_hbm.at[pl.ds(row0, CHUNK)], osem[s]).wait()


def _g_flat(groupings):
    gfl = groupings.reshape(-1).astype(jnp.int32)
    return jnp.pad(gfl, (0, RC_PAD - K))


def kernel(inputs, groupings):
    t = inputs.transpose(1, 0, 2).reshape(6, NBLK, 128, 7).transpose(0, 1, 3, 2)
    xrows = jnp.pad(t, ((0, 0), (0, 0), (0, 1), (0, 0))).reshape(6 * NBLK * 8, 128)
    out_rows = _row_gather_kernel(xrows, _g_flat(groupings))
    return out_rows.reshape(69, NBLK, 4, 128).transpose(1, 3, 0, 2).reshape(B, 69, 4)
